# baseline (speedup 1.0000x reference)
import functools

import jax
import jax.numpy as jnp
from jax import lax
from jax.experimental import pallas as pl
from jax.experimental.pallas import tpu as pltpu
from jax.experimental.pallas import tpu_sc as plsc

_ROWS = 16384
_D = 2048
_SC_ROWS = 4096
_TC_ROWS = _ROWS - _SC_ROWS
_TC_BLOCK = 512

_N_SC = _SC_ROWS * _D
_NW = 32
_PER_W = _N_SC // _NW
_CHUNK = 16384
_UNROLL = 8


def _tc_add_kernel(ts_ref, text_ref, out_ref):
    out_ref[...] = ts_ref[...] + text_ref[...]


def _tc_add(x2, y2):
    return pl.pallas_call(
        _tc_add_kernel,
        grid=(_TC_ROWS // _TC_BLOCK,),
        in_specs=[
            pl.BlockSpec((_TC_BLOCK, _D), lambda i: (i, 0)),
            pl.BlockSpec((_TC_BLOCK, _D), lambda i: (i, 0)),
        ],
        out_specs=pl.BlockSpec((_TC_BLOCK, _D), lambda i: (i, 0)),
        out_shape=jax.ShapeDtypeStruct((_ROWS, _D), x2.dtype),
    )(x2, y2)


@functools.partial(
    pl.kernel,
    mesh=plsc.VectorSubcoreMesh(core_axis_name="c", subcore_axis_name="s"),
    out_type=jax.ShapeDtypeStruct((_N_SC,), jnp.float32),
    scratch_types=[
        pltpu.VMEM((_CHUNK,), jnp.float32),
        pltpu.VMEM((_CHUNK,), jnp.float32),
    ],
)
def _sc_add(ts_hbm, text_hbm, out_hbm, a_v, b_v):
    wid = lax.axis_index("s") * 2 + lax.axis_index("c")
    base = _TC_ROWS * _D + wid * _PER_W

    def chunk_body(ci, carry):
        off = base + ci * _CHUNK
        pltpu.sync_copy(ts_hbm.at[pl.ds(off, _CHUNK)], a_v)
        pltpu.sync_copy(text_hbm.at[pl.ds(off, _CHUNK)], b_v)

        def add_body(i, c2):
            for u in range(_UNROLL):
                s = (i * _UNROLL + u) * 16
                a_v[pl.ds(s, 16)] = a_v[pl.ds(s, 16)] + b_v[pl.ds(s, 16)]
            return c2

        lax.fori_loop(0, _CHUNK // (16 * _UNROLL), add_body, 0)
        out_off = wid * _PER_W + ci * _CHUNK
        pltpu.sync_copy(a_v, out_hbm.at[pl.ds(out_off, _CHUNK)])
        return carry

    lax.fori_loop(0, _PER_W // _CHUNK, chunk_body, 0)


def kernel(ts, text, batch_idx):
    b, s, d = ts.shape
    x2 = ts.reshape(b * s, d)
    y2 = text.reshape(b * s, d)
    xf = ts.reshape(-1)
    yf = text.reshape(-1)

    tc_out = _tc_add(x2, y2)
    sc_out = _sc_add(xf, yf)

    out = lax.dynamic_update_slice(
        tc_out, sc_out.reshape(_SC_ROWS, _D), (_TC_ROWS, 0))
    aux_loss = jnp.zeros((), dtype=jnp.float32)
    return (out.reshape(b, s, d), aux_loss)


# speedup vs baseline: 2.3742x; 2.3742x over previous
import functools

import jax
import jax.numpy as jnp
from jax import lax
from jax.experimental import pallas as pl
from jax.experimental.pallas import tpu as pltpu
from jax.experimental.pallas import tpu_sc as plsc

_ROWS = 16384
_D = 2048
_SC_ROWS = 4096
_TC_ROWS = _ROWS - _SC_ROWS
_TC_BLOCK = 512

_NW = 32
_W_ROWS = _SC_ROWS // _NW
_CH_ROWS = 8
_N_CH = _W_ROWS // _CH_ROWS


def _tc_add_kernel(ts_ref, text_ref, out_ref):
    out_ref[...] = ts_ref[...] + text_ref[...]


def _tc_add(x2, y2):
    return pl.pallas_call(
        _tc_add_kernel,
        grid=(_TC_ROWS // _TC_BLOCK,),
        in_specs=[
            pl.BlockSpec((_TC_BLOCK, _D), lambda i: (i, 0)),
            pl.BlockSpec((_TC_BLOCK, _D), lambda i: (i, 0)),
        ],
        out_specs=pl.BlockSpec((_TC_BLOCK, _D), lambda i: (i, 0)),
        out_shape=jax.ShapeDtypeStruct((_ROWS, _D), x2.dtype),
    )(x2, y2)


@functools.partial(
    pl.kernel,
    mesh=plsc.VectorSubcoreMesh(core_axis_name="c", subcore_axis_name="s"),
    out_type=jax.ShapeDtypeStruct((_SC_ROWS, _D), jnp.float32),
    scratch_types=[
        pltpu.VMEM((_CH_ROWS, _D), jnp.float32),
        pltpu.VMEM((_CH_ROWS, _D), jnp.float32),
    ],
)
def _sc_add(ts_hbm, text_hbm, out_hbm, a_v, b_v):
    wid = lax.axis_index("s") * 2 + lax.axis_index("c")
    base = _TC_ROWS + wid * _W_ROWS

    def chunk_body(ci, carry):
        row = base + ci * _CH_ROWS
        pltpu.sync_copy(ts_hbm.at[pl.ds(row, _CH_ROWS)], a_v)
        pltpu.sync_copy(text_hbm.at[pl.ds(row, _CH_ROWS)], b_v)

        def add_body(i, c2):
            for u in range(8):
                c = (i * 8 + u) * 16
                for r in range(_CH_ROWS):
                    a_v[r, pl.ds(c, 16)] = a_v[r, pl.ds(c, 16)] + b_v[r, pl.ds(c, 16)]
            return c2

        lax.fori_loop(0, _D // (16 * 8), add_body, 0)
        out_row = wid * _W_ROWS + ci * _CH_ROWS
        pltpu.sync_copy(a_v, out_hbm.at[pl.ds(out_row, _CH_ROWS)])
        return carry

    lax.fori_loop(0, _N_CH, chunk_body, 0)


def _merge_kernel(dst_ref, src_ref, out_ref):
    del dst_ref
    out_ref[...] = src_ref[...]


def _merge(tc_out, sc_out):
    tc_blocks = _TC_ROWS // _TC_BLOCK
    return pl.pallas_call(
        _merge_kernel,
        grid=(_SC_ROWS // _TC_BLOCK,),
        in_specs=[
            pl.BlockSpec((8, 128), lambda i: (0, 0)),
            pl.BlockSpec((_TC_BLOCK, _D), lambda i: (i, 0)),
        ],
        out_specs=pl.BlockSpec((_TC_BLOCK, _D), lambda i: (i + tc_blocks, 0)),
        out_shape=jax.ShapeDtypeStruct((_ROWS, _D), tc_out.dtype),
        input_output_aliases={0: 0},
    )(tc_out, sc_out)


def kernel(ts, text, batch_idx):
    b, s, d = ts.shape
    x2 = ts.reshape(b * s, d)
    y2 = text.reshape(b * s, d)

    tc_out = _tc_add(x2, y2)
    sc_out = _sc_add(x2, y2)

    out = _merge(tc_out, sc_out)
    aux_loss = jnp.zeros((), dtype=jnp.float32)
    return (out.reshape(b, s, d), aux_loss)


# speedup vs baseline: 2.5539x; 1.0757x over previous
import functools

import jax
import jax.numpy as jnp
from jax import lax
from jax.experimental import pallas as pl
from jax.experimental.pallas import tpu as pltpu
from jax.experimental.pallas import tpu_sc as plsc

_ROWS = 16384
_D = 2048
_SC_ROWS = 2048
_TC_ROWS = _ROWS - _SC_ROWS
_TC_BLOCK = 512

_NW = 32
_W_ROWS = _SC_ROWS // _NW
_CH_ROWS = 8
_N_CH = _W_ROWS // _CH_ROWS


def _tc_add_kernel(ts_ref, text_ref, out_ref):
    out_ref[...] = ts_ref[...] + text_ref[...]


def _tc_add(x2, y2):
    return pl.pallas_call(
        _tc_add_kernel,
        grid=(_TC_ROWS // _TC_BLOCK,),
        in_specs=[
            pl.BlockSpec((_TC_BLOCK, _D), lambda i: (i, 0)),
            pl.BlockSpec((_TC_BLOCK, _D), lambda i: (i, 0)),
        ],
        out_specs=pl.BlockSpec((_TC_BLOCK, _D), lambda i: (i, 0)),
        out_shape=jax.ShapeDtypeStruct((_ROWS, _D), x2.dtype),
    )(x2, y2)


@functools.partial(
    pl.kernel,
    mesh=plsc.VectorSubcoreMesh(core_axis_name="c", subcore_axis_name="s"),
    out_type=jax.ShapeDtypeStruct((_SC_ROWS, _D), jnp.float32),
    scratch_types=[
        pltpu.VMEM((_CH_ROWS, _D), jnp.float32),
        pltpu.VMEM((_CH_ROWS, _D), jnp.float32),
    ],
)
def _sc_add(ts_hbm, text_hbm, out_hbm, a_v, b_v):
    wid = lax.axis_index("s") * 2 + lax.axis_index("c")
    base = _TC_ROWS + wid * _W_ROWS

    def chunk_body(ci, carry):
        row = base + ci * _CH_ROWS
        pltpu.sync_copy(ts_hbm.at[pl.ds(row, _CH_ROWS)], a_v)
        pltpu.sync_copy(text_hbm.at[pl.ds(row, _CH_ROWS)], b_v)

        def add_body(i, c2):
            for u in range(8):
                c = (i * 8 + u) * 16
                for r in range(_CH_ROWS):
                    a_v[r, pl.ds(c, 16)] = a_v[r, pl.ds(c, 16)] + b_v[r, pl.ds(c, 16)]
            return c2

        lax.fori_loop(0, _D // (16 * 8), add_body, 0)
        out_row = wid * _W_ROWS + ci * _CH_ROWS
        pltpu.sync_copy(a_v, out_hbm.at[pl.ds(out_row, _CH_ROWS)])
        return carry

    lax.fori_loop(0, _N_CH, chunk_body, 0)


def _merge_kernel(dst_ref, src_ref, out_ref):
    del dst_ref
    out_ref[...] = src_ref[...]


def _merge(tc_out, sc_out):
    tc_blocks = _TC_ROWS // _TC_BLOCK
    return pl.pallas_call(
        _merge_kernel,
        grid=(_SC_ROWS // _TC_BLOCK,),
        in_specs=[
            pl.BlockSpec((8, 128), lambda i: (0, 0)),
            pl.BlockSpec((_TC_BLOCK, _D), lambda i: (i, 0)),
        ],
        out_specs=pl.BlockSpec((_TC_BLOCK, _D), lambda i: (i + tc_blocks, 0)),
        out_shape=jax.ShapeDtypeStruct((_ROWS, _D), tc_out.dtype),
        input_output_aliases={0: 0},
    )(tc_out, sc_out)


def kernel(ts, text, batch_idx):
    b, s, d = ts.shape
    x2 = ts.reshape(b * s, d)
    y2 = text.reshape(b * s, d)

    tc_out = _tc_add(x2, y2)
    sc_out = _sc_add(x2, y2)

    out = _merge(tc_out, sc_out)
    aux_loss = jnp.zeros((), dtype=jnp.float32)
    return (out.reshape(b, s, d), aux_loss)


# speedup vs baseline: 2.7103x; 1.0612x over previous
import functools

import jax
import jax.numpy as jnp
from jax import lax
from jax.experimental import pallas as pl
from jax.experimental.pallas import tpu as pltpu
from jax.experimental.pallas import tpu_sc as plsc

_ROWS = 16384
_D = 2048
_SC_ROWS = 512
_TC_ROWS = _ROWS - _SC_ROWS
_TC_BLOCK = 512

_NW = 32
_W_ROWS = _SC_ROWS // _NW
_CH_ROWS = 8
_N_CH = _W_ROWS // _CH_ROWS


def _tc_add_kernel(ts_ref, text_ref, out_ref):
    out_ref[...] = ts_ref[...] + text_ref[...]


def _tc_add(x2, y2):
    return pl.pallas_call(
        _tc_add_kernel,
        grid=(_TC_ROWS // _TC_BLOCK,),
        in_specs=[
            pl.BlockSpec((_TC_BLOCK, _D), lambda i: (i, 0)),
            pl.BlockSpec((_TC_BLOCK, _D), lambda i: (i, 0)),
        ],
        out_specs=pl.BlockSpec((_TC_BLOCK, _D), lambda i: (i, 0)),
        out_shape=jax.ShapeDtypeStruct((_ROWS, _D), x2.dtype),
    )(x2, y2)


@functools.partial(
    pl.kernel,
    mesh=plsc.VectorSubcoreMesh(core_axis_name="c", subcore_axis_name="s"),
    out_type=jax.ShapeDtypeStruct((_SC_ROWS, _D), jnp.float32),
    scratch_types=[
        pltpu.VMEM((_CH_ROWS, _D), jnp.float32),
        pltpu.VMEM((_CH_ROWS, _D), jnp.float32),
    ],
)
def _sc_add(ts_hbm, text_hbm, out_hbm, a_v, b_v):
    wid = lax.axis_index("s") * 2 + lax.axis_index("c")
    base = _TC_ROWS + wid * _W_ROWS

    def chunk_body(ci, carry):
        row = base + ci * _CH_ROWS
        pltpu.sync_copy(ts_hbm.at[pl.ds(row, _CH_ROWS)], a_v)
        pltpu.sync_copy(text_hbm.at[pl.ds(row, _CH_ROWS)], b_v)

        def add_body(i, c2):
            for u in range(8):
                c = (i * 8 + u) * 16
                for r in range(_CH_ROWS):
                    a_v[r, pl.ds(c, 16)] = a_v[r, pl.ds(c, 16)] + b_v[r, pl.ds(c, 16)]
            return c2

        lax.fori_loop(0, _D // (16 * 8), add_body, 0)
        out_row = wid * _W_ROWS + ci * _CH_ROWS
        pltpu.sync_copy(a_v, out_hbm.at[pl.ds(out_row, _CH_ROWS)])
        return carry

    lax.fori_loop(0, _N_CH, chunk_body, 0)


def _merge_kernel(dst_ref, src_ref, out_ref):
    del dst_ref
    out_ref[...] = src_ref[...]


def _merge(tc_out, sc_out):
    tc_blocks = _TC_ROWS // _TC_BLOCK
    return pl.pallas_call(
        _merge_kernel,
        grid=(_SC_ROWS // _TC_BLOCK,),
        in_specs=[
            pl.BlockSpec((8, 128), lambda i: (0, 0)),
            pl.BlockSpec((_TC_BLOCK, _D), lambda i: (i, 0)),
        ],
        out_specs=pl.BlockSpec((_TC_BLOCK, _D), lambda i: (i + tc_blocks, 0)),
        out_shape=jax.ShapeDtypeStruct((_ROWS, _D), tc_out.dtype),
        input_output_aliases={0: 0},
    )(tc_out, sc_out)


def kernel(ts, text, batch_idx):
    b, s, d = ts.shape
    x2 = ts.reshape(b * s, d)
    y2 = text.reshape(b * s, d)

    tc_out = _tc_add(x2, y2)
    sc_out = _sc_add(x2, y2)

    out = _merge(tc_out, sc_out)
    aux_loss = jnp.zeros((), dtype=jnp.float32)
    return (out.reshape(b, s, d), aux_loss)


# speedup vs baseline: 3.1463x; 1.1609x over previous
import jax
import jax.numpy as jnp
from jax.experimental import pallas as pl


def _add_kernel(ts_ref, text_ref, out_ref):
    out_ref[...] = ts_ref[...] + text_ref[...]


def kernel(ts, text, batch_idx):
    b, s, d = ts.shape
    x2 = ts.reshape(b * s, d)
    y2 = text.reshape(b * s, d)
    rows = b * s
    block_rows = 512
    grid = (rows // block_rows,)
    out = pl.pallas_call(
        _add_kernel,
        grid=grid,
        in_specs=[
            pl.BlockSpec((block_rows, d), lambda i: (i, 0)),
            pl.BlockSpec((block_rows, d), lambda i: (i, 0)),
        ],
        out_specs=pl.BlockSpec((block_rows, d), lambda i: (i, 0)),
        out_shape=jax.ShapeDtypeStruct((rows, d), ts.dtype),
    )(x2, y2)
    aux_loss = jnp.zeros((), dtype=jnp.float32)
    return (out.reshape(b, s, d), aux_loss)
